# Initial kernel scaffold; baseline (speedup 1.0000x reference)
import jax
import jax.numpy as jnp
from jax.experimental import pallas as pl


def kernel(inputs_0, inputs_1, inputs_2, emb, emb_w, emb_b):
    raise NotImplementedError("write your pallas kernel here")


# speedup vs baseline: 3.7919x; 3.7919x over previous
import jax
import jax.numpy as jnp
from jax import lax
from jax.experimental import pallas as pl
from jax.experimental.pallas import tpu as pltpu
from jax.experimental.pallas import tpu_sc as plsc

VOCAB = 1000000
EMB_DIM = 64
NEG_NUM = 64
BATCH = 4096

NC = 2
NS = 16
NW = NC * NS
BPW = BATCH // NW


def _w2v_body(in0_hbm, in1_hbm, in2_hbm, emb_hbm, embw_hbm, embb_hbm,
              outt_hbm, outn_hbm,
              idx0_v, idx1_v, idx2_v, inp_v, tw_v, tb_v, w_v, nb_v,
              outt_v, outn_v):
    wid = lax.axis_index("s") * NC + lax.axis_index("c")
    base = wid * BPW

    iota = lax.iota(jnp.int32, 16)
    zeros16 = jnp.zeros((16,), jnp.int32)
    rows = [iota + (16 * g) for g in range(4)]
    cols = [jnp.full((16,), d, jnp.int32) for d in range(EMB_DIM)]

    pltpu.sync_copy(in0_hbm.at[pl.ds(base, BPW)], idx0_v)
    pltpu.sync_copy(in1_hbm.at[pl.ds(base, BPW)], idx1_v)
    pltpu.sync_copy(in2_hbm.at[pl.ds(base, BPW), :], idx2_v)

    pltpu.sync_copy(emb_hbm.at[idx0_v], inp_v)
    pltpu.sync_copy(embw_hbm.at[idx1_v], tw_v)
    pltpu.sync_copy(embb_hbm.at[idx1_v], tb_v)

    def t_body(grp, carry):
        brows = iota + grp * 16
        acc = tb_v[pl.ds(grp * 16, 16)]
        for d in range(EMB_DIM):
            a = plsc.load_gather(inp_v, [brows, cols[d]])
            t = plsc.load_gather(tw_v, [brows, cols[d]])
            acc = acc + a * t
        outt_v[pl.ds(grp * 16, 16)] = acc
        return carry

    lax.fori_loop(0, BPW // 16, t_body, 0)

    def n_body(b, carry):
        pltpu.sync_copy(embw_hbm.at[idx2_v.at[b]], w_v)
        pltpu.sync_copy(embb_hbm.at[idx2_v.at[b]], nb_v)
        inp_c = [inp_v[b, pl.ds(16 * c, 16)] for c in range(4)]
        accs = [nb_v[pl.ds(16 * g, 16)] for g in range(4)]
        for k in range(NEG_NUM):
            s = inp_c[k // 16][k % 16]
            for g in range(4):
                accs[g] = accs[g] + w_v[k, pl.ds(16 * g, 16)] * s
        for g in range(4):
            outn_v[b, pl.ds(16 * g, 16)] = accs[g]
        return carry

    lax.fori_loop(0, BPW, n_body, 0)

    pltpu.sync_copy(outt_v, outt_hbm.at[pl.ds(base, BPW)])
    pltpu.sync_copy(outn_v, outn_hbm.at[pl.ds(base, BPW), :])


@jax.jit
def _w2v(in0, in1, in2, emb, emb_w, emb_b):
    mesh = plsc.VectorSubcoreMesh(core_axis_name="c", subcore_axis_name="s")
    f = pl.kernel(
        _w2v_body,
        out_type=(
            jax.ShapeDtypeStruct((BATCH,), jnp.float32),
            jax.ShapeDtypeStruct((BATCH, NEG_NUM), jnp.float32),
        ),
        mesh=mesh,
        compiler_params=pltpu.CompilerParams(
            needs_layout_passes=False, use_tc_tiling_on_sc=False),
        scratch_types=[
            pltpu.VMEM((BPW,), jnp.int32),
            pltpu.VMEM((BPW,), jnp.int32),
            pltpu.VMEM((BPW, NEG_NUM), jnp.int32),
            pltpu.VMEM((BPW, EMB_DIM), jnp.float32),
            pltpu.VMEM((BPW, EMB_DIM), jnp.float32),
            pltpu.VMEM((BPW,), jnp.float32),
            pltpu.VMEM((NEG_NUM, EMB_DIM), jnp.float32),
            pltpu.VMEM((NEG_NUM,), jnp.float32),
            pltpu.VMEM((BPW,), jnp.float32),
            pltpu.VMEM((BPW, NEG_NUM), jnp.float32),
        ],
    )
    return f(in0, in1, in2, emb, emb_w, emb_b)


def kernel(inputs_0, inputs_1, inputs_2, emb, emb_w, emb_b):
    in0 = inputs_0.reshape(BATCH).astype(jnp.int32)
    in1 = inputs_1.reshape(BATCH).astype(jnp.int32)
    in2 = inputs_2.astype(jnp.int32)
    true_flat, neg_logits = _w2v(in0, in1, in2, emb, emb_w,
                                 emb_b.reshape(VOCAB))
    return true_flat.reshape(BATCH, 1), neg_logits


# speedup vs baseline: 4.2977x; 1.1334x over previous
import jax
import jax.numpy as jnp
from jax import lax
from jax.experimental import pallas as pl
from jax.experimental.pallas import tpu as pltpu
from jax.experimental.pallas import tpu_sc as plsc

VOCAB = 1000000
EMB_DIM = 64
NEG_NUM = 64
BATCH = 4096

NC = 2
NS = 16
NW = NC * NS
BPW = BATCH // NW

CHUNK = 2
NBUF = 2
NCHUNK = BPW // CHUNK


def _w2v_body(in0_hbm, in1_hbm, in2_hbm, emb_hbm, embw_hbm, embb_hbm,
              outt_hbm, outn_hbm,
              idx0_v, idx1_v, idx2_v, inp_v, tw_v, tb_v,
              w_bufs, nb_bufs, outt_v, outn_v, wsem, nbsem):
    wid = lax.axis_index("s") * NC + lax.axis_index("c")
    base = wid * BPW

    iota = lax.iota(jnp.int32, 16)
    cols = [jnp.full((16,), d, jnp.int32) for d in range(EMB_DIM)]

    pltpu.sync_copy(in0_hbm.at[pl.ds(base, BPW)], idx0_v)
    pltpu.sync_copy(in1_hbm.at[pl.ds(base, BPW)], idx1_v)
    pltpu.sync_copy(in2_hbm.at[pl.ds(wid * NCHUNK, NCHUNK), :], idx2_v)

    def start_chunk(c, p):
        pltpu.async_copy(embw_hbm.at[idx2_v.at[c]], w_bufs.at[p], wsem.at[p])
        pltpu.async_copy(embb_hbm.at[idx2_v.at[c]], nb_bufs.at[p], nbsem.at[p])

    def wait_chunk(c, p):
        pltpu.make_async_copy(
            embw_hbm.at[idx2_v.at[c]], w_bufs.at[p], wsem.at[p]).wait()
        pltpu.make_async_copy(
            embb_hbm.at[idx2_v.at[c]], nb_bufs.at[p], nbsem.at[p]).wait()

    for p in range(NBUF):
        start_chunk(p, p)

    pltpu.sync_copy(emb_hbm.at[idx0_v], inp_v)
    pltpu.sync_copy(embw_hbm.at[idx1_v], tw_v)
    pltpu.sync_copy(embb_hbm.at[idx1_v], tb_v)

    def t_body(grp, carry):
        brows = iota + grp * 16
        acc = tb_v[pl.ds(grp * 16, 16)]
        for d in range(EMB_DIM):
            a = plsc.load_gather(inp_v, [brows, cols[d]])
            t = plsc.load_gather(tw_v, [brows, cols[d]])
            acc = acc + a * t
        outt_v[pl.ds(grp * 16, 16)] = acc
        return carry

    lax.fori_loop(0, BPW // 16, t_body, 0)

    def n_body(i, carry):
        for p in range(NBUF):
            c = i * NBUF + p
            wait_chunk(c, p)
            for cb in range(CHUNK):
                b = c * CHUNK + cb
                inp_c = [inp_v[b, pl.ds(16 * h, 16)] for h in range(4)]
                accs = [nb_bufs[p, pl.ds(cb * NEG_NUM + 16 * g, 16)]
                        for g in range(4)]
                for k in range(NEG_NUM):
                    s = inp_c[k // 16][k % 16]
                    row = cb * NEG_NUM + k
                    for g in range(4):
                        accs[g] = accs[g] + w_bufs[p, row, pl.ds(16 * g, 16)] * s
                for g in range(4):
                    outn_v[b, pl.ds(16 * g, 16)] = accs[g]

            @pl.when(c + NBUF < NCHUNK)
            def _():
                start_chunk(c + NBUF, p)
        return carry

    lax.fori_loop(0, NCHUNK // NBUF, n_body, 0)

    pltpu.sync_copy(outt_v, outt_hbm.at[pl.ds(base, BPW)])
    pltpu.sync_copy(outn_v, outn_hbm.at[pl.ds(base, BPW), :])


@jax.jit
def _w2v(in0, in1, in2, emb, emb_w, emb_b):
    mesh = plsc.VectorSubcoreMesh(core_axis_name="c", subcore_axis_name="s")
    f = pl.kernel(
        _w2v_body,
        out_type=(
            jax.ShapeDtypeStruct((BATCH,), jnp.float32),
            jax.ShapeDtypeStruct((BATCH, NEG_NUM), jnp.float32),
        ),
        mesh=mesh,
        compiler_params=pltpu.CompilerParams(
            needs_layout_passes=False, use_tc_tiling_on_sc=False),
        scratch_types=[
            pltpu.VMEM((BPW,), jnp.int32),
            pltpu.VMEM((BPW,), jnp.int32),
            pltpu.VMEM((NCHUNK, CHUNK * NEG_NUM), jnp.int32),
            pltpu.VMEM((BPW, EMB_DIM), jnp.float32),
            pltpu.VMEM((BPW, EMB_DIM), jnp.float32),
            pltpu.VMEM((BPW,), jnp.float32),
            pltpu.VMEM((NBUF, CHUNK * NEG_NUM, EMB_DIM), jnp.float32),
            pltpu.VMEM((NBUF, CHUNK * NEG_NUM), jnp.float32),
            pltpu.VMEM((BPW,), jnp.float32),
            pltpu.VMEM((BPW, NEG_NUM), jnp.float32),
            pltpu.SemaphoreType.DMA((NBUF,)),
            pltpu.SemaphoreType.DMA((NBUF,)),
        ],
    )
    return f(in0, in1, in2, emb, emb_w, emb_b)


def kernel(inputs_0, inputs_1, inputs_2, emb, emb_w, emb_b):
    in0 = inputs_0.reshape(BATCH).astype(jnp.int32)
    in1 = inputs_1.reshape(BATCH).astype(jnp.int32)
    in2 = inputs_2.astype(jnp.int32).reshape(BATCH // CHUNK, CHUNK * NEG_NUM)
    true_flat, neg_logits = _w2v(in0, in1, in2, emb, emb_w,
                                 emb_b.reshape(VOCAB))
    return true_flat.reshape(BATCH, 1), neg_logits


# speedup vs baseline: 7.2584x; 1.6889x over previous
import jax
import jax.numpy as jnp
from jax import lax
from jax.experimental import pallas as pl
from jax.experimental.pallas import tpu as pltpu
from jax.experimental.pallas import tpu_sc as plsc

VOCAB = 1000000
EMB_DIM = 64
NEG_NUM = 64
BATCH = 4096

NC = 2
NS = 16
NW = NC * NS
BPW = BATCH // NW

CHUNK = 2
NBUF = 2
NCHUNK = BPW // CHUNK


def _gather_inp_body(embT_hbm, idx_hbm, out_hbm, idx_v, bbufs, rows_v, sems):
    wid = lax.axis_index("s") * NC + lax.axis_index("c")
    base = wid * BPW

    iota = lax.iota(jnp.int32, 16)
    pltpu.sync_copy(idx_hbm.at[pl.ds(base, BPW)], idx_v)

    def issue(v, p):
        cb = (v // 128) * 128
        pltpu.async_copy(embT_hbm.at[:, pl.ds(cb, 128)], bbufs.at[p],
                         sems.at[p])

    def wait(v, p):
        cb = (v // 128) * 128
        pltpu.make_async_copy(embT_hbm.at[:, pl.ds(cb, 128)], bbufs.at[p],
                              sems.at[p]).wait()

    first = idx_v[pl.ds(0, 16)]
    issue(first[0], 0)

    def chunk_body(ci, carry):
        vec = idx_v[pl.ds(ci * 16, 16)]
        nci = jnp.minimum(ci + 1, (BPW // 16) - 1)
        nvec = idx_v[pl.ds(nci * 16, 16)]
        for lane in range(16):
            p = lane % 2
            v = vec[lane]
            if lane + 1 < 16:
                issue(vec[lane + 1], 1 - p)
            else:
                @pl.when(ci + 1 < BPW // 16)
                def _():
                    issue(nvec[0], 1 - p)
            wait(v, p)
            col = jnp.full((16,), v % 128, jnp.int32)
            i = ci * 16 + lane
            for g in range(4):
                w = plsc.load_gather(bbufs.at[p], [iota + 16 * g, col])
                rows_v[pl.ds(i * EMB_DIM + 16 * g, 16)] = w
        return carry

    lax.fori_loop(0, BPW // 16, chunk_body, 0)
    pltpu.sync_copy(rows_v, out_hbm.at[pl.ds(base * EMB_DIM, BPW * EMB_DIM)])


def _w2v_body(in1_hbm, in2_hbm, inp_hbm, embw_hbm, embb_hbm,
              outt_hbm, outn_hbm,
              idx1_v, idx2_v, inp_vf, tw_v, tb_v,
              w_bufs, nb_bufs, outt_v, outn_v, wsem, nbsem):
    wid = lax.axis_index("s") * NC + lax.axis_index("c")
    base = wid * BPW

    iota = lax.iota(jnp.int32, 16)
    cols = [jnp.full((16,), d, jnp.int32) for d in range(EMB_DIM)]

    pltpu.sync_copy(in1_hbm.at[pl.ds(base, BPW)], idx1_v)
    pltpu.sync_copy(in2_hbm.at[pl.ds(wid * NCHUNK, NCHUNK), :], idx2_v)

    def start_chunk(c, p):
        pltpu.async_copy(embw_hbm.at[idx2_v.at[c]], w_bufs.at[p], wsem.at[p])
        pltpu.async_copy(embb_hbm.at[idx2_v.at[c]], nb_bufs.at[p], nbsem.at[p])

    def wait_chunk(c, p):
        pltpu.make_async_copy(
            embw_hbm.at[idx2_v.at[c]], w_bufs.at[p], wsem.at[p]).wait()
        pltpu.make_async_copy(
            embb_hbm.at[idx2_v.at[c]], nb_bufs.at[p], nbsem.at[p]).wait()

    for p in range(NBUF):
        start_chunk(p, p)

    pltpu.sync_copy(inp_hbm.at[pl.ds(base * EMB_DIM, BPW * EMB_DIM)], inp_vf)
    pltpu.sync_copy(embw_hbm.at[idx1_v], tw_v)
    pltpu.sync_copy(embb_hbm.at[idx1_v], tb_v)

    def t_body(grp, carry):
        brows = iota + grp * 16
        brows64 = brows * EMB_DIM
        acc = tb_v[pl.ds(grp * 16, 16)]
        for d in range(EMB_DIM):
            a = plsc.load_gather(inp_vf, [brows64 + cols[d]])
            t = plsc.load_gather(tw_v, [brows, cols[d]])
            acc = acc + a * t
        outt_v[pl.ds(grp * 16, 16)] = acc
        return carry

    lax.fori_loop(0, BPW // 16, t_body, 0)

    def n_body(i, carry):
        for p in range(NBUF):
            c = i * NBUF + p
            wait_chunk(c, p)
            for cb in range(CHUNK):
                b = c * CHUNK + cb
                inp_c = [inp_vf[pl.ds(b * EMB_DIM + 16 * h, 16)]
                         for h in range(4)]
                accs = [nb_bufs[p, pl.ds(cb * NEG_NUM + 16 * g, 16)]
                        for g in range(4)]
                for k in range(NEG_NUM):
                    s = inp_c[k // 16][k % 16]
                    row = cb * NEG_NUM + k
                    for g in range(4):
                        accs[g] = accs[g] + w_bufs[p, row, pl.ds(16 * g, 16)] * s
                for g in range(4):
                    outn_v[b, pl.ds(16 * g, 16)] = accs[g]

            @pl.when(c + NBUF < NCHUNK)
            def _():
                start_chunk(c + NBUF, p)
        return carry

    lax.fori_loop(0, NCHUNK // NBUF, n_body, 0)

    pltpu.sync_copy(outt_v, outt_hbm.at[pl.ds(base, BPW)])
    pltpu.sync_copy(outn_v, outn_hbm.at[pl.ds(base, BPW), :])


@jax.jit
def _w2v(in0, in1, in2, emb, emb_w, emb_b):
    mesh = plsc.VectorSubcoreMesh(core_axis_name="c", subcore_axis_name="s")

    gather_inp = pl.kernel(
        _gather_inp_body,
        out_type=jax.ShapeDtypeStruct((BATCH * EMB_DIM,), jnp.float32),
        mesh=mesh,
        compiler_params=pltpu.CompilerParams(
            needs_layout_passes=False, use_tc_tiling_on_sc=True),
        scratch_types=[
            pltpu.VMEM((BPW,), jnp.int32),
            pltpu.VMEM((2, EMB_DIM, 128), jnp.float32),
            pltpu.VMEM((BPW * EMB_DIM,), jnp.float32),
            pltpu.SemaphoreType.DMA((2,)),
        ],
    )
    inp_flat = gather_inp(emb.T, in0)

    f = pl.kernel(
        _w2v_body,
        out_type=(
            jax.ShapeDtypeStruct((BATCH,), jnp.float32),
            jax.ShapeDtypeStruct((BATCH, NEG_NUM), jnp.float32),
        ),
        mesh=mesh,
        compiler_params=pltpu.CompilerParams(
            needs_layout_passes=False, use_tc_tiling_on_sc=False),
        scratch_types=[
            pltpu.VMEM((BPW,), jnp.int32),
            pltpu.VMEM((NCHUNK, CHUNK * NEG_NUM), jnp.int32),
            pltpu.VMEM((BPW * EMB_DIM,), jnp.float32),
            pltpu.VMEM((BPW, EMB_DIM), jnp.float32),
            pltpu.VMEM((BPW,), jnp.float32),
            pltpu.VMEM((NBUF, CHUNK * NEG_NUM, EMB_DIM), jnp.float32),
            pltpu.VMEM((NBUF, CHUNK * NEG_NUM), jnp.float32),
            pltpu.VMEM((BPW,), jnp.float32),
            pltpu.VMEM((BPW, NEG_NUM), jnp.float32),
            pltpu.SemaphoreType.DMA((NBUF,)),
            pltpu.SemaphoreType.DMA((NBUF,)),
        ],
    )
    return f(in1, in2, inp_flat, emb_w, emb_b)


def kernel(inputs_0, inputs_1, inputs_2, emb, emb_w, emb_b):
    in0 = inputs_0.reshape(BATCH).astype(jnp.int32)
    in1 = inputs_1.reshape(BATCH).astype(jnp.int32)
    in2 = inputs_2.astype(jnp.int32).reshape(BATCH // CHUNK, CHUNK * NEG_NUM)
    true_flat, neg_logits = _w2v(in0, in1, in2, emb, emb_w,
                                 emb_b.reshape(VOCAB))
    return true_flat.reshape(BATCH, 1), neg_logits
